# Initial kernel scaffold; baseline (speedup 1.0000x reference)
import jax
import jax.numpy as jnp
from jax.experimental import pallas as pl


def kernel(x, edge_index, W1, b1, W2, b2):
    raise NotImplementedError("write your pallas kernel here")


# speedup vs baseline: 17.8026x; 17.8026x over previous
import functools

import jax
import jax.numpy as jnp
from jax import lax
from jax.experimental import pallas as pl
from jax.experimental.pallas import tpu as pltpu
from jax.experimental.pallas import tpu_sc as plsc

N = 10000
E = 320000
D = 128

NC = 2
NS = 16
NW = NC * NS
EPW = E // NW
CH = 80
NCHUNK = EPW // CH
RPT = 624
TAIL = N - NS * RPT
ACC_ROWS = 10240
ZPT = ACC_ROWS // NS
DEGW = 128


def _sc_mesh():
    return plsc.VectorSubcoreMesh(
        core_axis_name="c", subcore_axis_name="s", num_cores=NC,
        num_subcores=NS)


def _deg_body(dst_hbm, ones_hbm, zer_hbm, out_hbm, didx_v, ones_v, zer_v,
              dacc_sh):
    c = lax.axis_index("c")
    s = lax.axis_index("s")
    w = s * NC + c
    pltpu.sync_copy(dst_hbm.at[w], didx_v)
    pltpu.sync_copy(ones_hbm, ones_v)
    pltpu.sync_copy(zer_hbm, zer_v)

    def zero_k(k, _):
        pltpu.sync_copy(zer_v, dacc_sh.at[pl.ds(s * ZPT + k * CH, CH)])
        return _
    lax.fori_loop(0, ZPT // CH, zero_k, None)
    plsc.subcore_barrier()

    def body(j, _):
        pltpu.sync_copy(ones_v, dacc_sh.at[didx_v.at[j]], add=True)
        return _
    lax.fori_loop(0, NCHUNK, body, None)
    plsc.subcore_barrier()
    pltpu.sync_copy(dacc_sh.at[pl.ds(s * RPT, RPT)],
                    out_hbm.at[c].at[pl.ds(s * RPT, RPT)])

    @pl.when(s == NS - 1)
    def _tail():
        pltpu.sync_copy(dacc_sh.at[pl.ds(NS * RPT, TAIL)],
                        out_hbm.at[c].at[pl.ds(NS * RPT, TAIL)])


def _deg(dst_r, ones8, zer8):
    k = pl.kernel(
        _deg_body,
        out_type=jax.ShapeDtypeStruct((NC, N, DEGW), jnp.float32),
        mesh=_sc_mesh(),
        scratch_types=[
            pltpu.VMEM((NCHUNK, CH), jnp.int32),
            pltpu.VMEM((CH, DEGW), jnp.float32),
            pltpu.VMEM((CH, DEGW), jnp.float32),
            pltpu.VMEM_SHARED((ACC_ROWS, DEGW), jnp.float32),
        ],
    )
    return k(dst_r, ones8, zer8)


def _scat_body(h_hbm, src_hbm, dst_hbm, zer_hbm, out_hbm, sidx_v, didx_v,
               rows_v, acc_sh, sem):
    c = lax.axis_index("c")
    s = lax.axis_index("s")
    w = s * NC + c
    pltpu.sync_copy(src_hbm.at[w], sidx_v)
    pltpu.sync_copy(dst_hbm.at[w], didx_v)
    pltpu.sync_copy(zer_hbm, rows_v)

    def zero_k(k, _):
        pltpu.sync_copy(rows_v, acc_sh.at[pl.ds(s * ZPT + k * CH, CH)])
        return _
    lax.fori_loop(0, ZPT // CH, zero_k, None)
    plsc.subcore_barrier()

    def body(j, _):
        pltpu.async_copy(h_hbm.at[sidx_v.at[j]], rows_v, sem).wait()
        pltpu.sync_copy(rows_v, acc_sh.at[didx_v.at[j]], add=True)
        return _
    lax.fori_loop(0, NCHUNK, body, None)
    plsc.subcore_barrier()
    pltpu.sync_copy(acc_sh.at[pl.ds(s * RPT, RPT)],
                    out_hbm.at[c].at[pl.ds(s * RPT, RPT)])

    @pl.when(s == NS - 1)
    def _tail():
        pltpu.sync_copy(acc_sh.at[pl.ds(NS * RPT, TAIL)],
                        out_hbm.at[c].at[pl.ds(NS * RPT, TAIL)])


def _scatter(h, src_r, dst_r, zer128):
    k = pl.kernel(
        _scat_body,
        out_type=jax.ShapeDtypeStruct((NC, N, D), jnp.float32),
        mesh=_sc_mesh(),
        scratch_types=[
            pltpu.VMEM((NCHUNK, CH), jnp.int32),
            pltpu.VMEM((NCHUNK, CH), jnp.int32),
            pltpu.VMEM((CH, D), jnp.float32),
            pltpu.VMEM_SHARED((ACC_ROWS, D), jnp.float32),
            pltpu.SemaphoreType.DMA,
        ],
    )
    return k(h, src_r, dst_r, zer128)


_BM = 1000


def _dis(dref):
    deg = dref[0, :, 0:1] + dref[1, :, 0:1] + 1.0
    return lax.rsqrt(deg)


def _tc1_body(x_ref, w_ref, d_ref, o_ref):
    o_ref[...] = jnp.dot(x_ref[...], w_ref[...],
                         preferred_element_type=jnp.float32) * _dis(d_ref)


def _tc_mid_body(p_ref, h_ref, d_ref, b_ref, w_ref, o_ref):
    dis = _dis(d_ref)
    t = (p_ref[0] + p_ref[1] + h_ref[...]) * dis + b_ref[...]
    t = jnp.maximum(t, 0.0)
    o_ref[...] = jnp.dot(t, w_ref[...],
                         preferred_element_type=jnp.float32) * dis


def _tc_fin_body(p_ref, h_ref, d_ref, b_ref, o_ref):
    o_ref[...] = ((p_ref[0] + p_ref[1] + h_ref[...]) * _dis(d_ref)
                  + b_ref[...])


def _row_specs():
    return dict(
        p=pl.BlockSpec((NC, _BM, D), lambda i: (0, i, 0)),
        h=pl.BlockSpec((_BM, D), lambda i: (i, 0)),
        d=pl.BlockSpec((NC, _BM, DEGW), lambda i: (0, i, 0)),
        b=pl.BlockSpec((1, D), lambda i: (0, 0)),
        w=pl.BlockSpec((D, D), lambda i: (0, 0)),
    )


def _tc1(x, w1t, degp):
    sp = _row_specs()
    return pl.pallas_call(
        _tc1_body,
        grid=(N // _BM,),
        in_specs=[sp["h"], sp["w"], sp["d"]],
        out_specs=sp["h"],
        out_shape=jax.ShapeDtypeStruct((N, D), jnp.float32),
    )(x, w1t, degp)


def _tc_mid(part, h1p, degp, b1, w2t):
    sp = _row_specs()
    return pl.pallas_call(
        _tc_mid_body,
        grid=(N // _BM,),
        in_specs=[sp["p"], sp["h"], sp["d"], sp["b"], sp["w"]],
        out_specs=sp["h"],
        out_shape=jax.ShapeDtypeStruct((N, D), jnp.float32),
    )(part, h1p, degp, b1, w2t)


def _tc_fin(part, h2p, degp, b2):
    sp = _row_specs()
    return pl.pallas_call(
        _tc_fin_body,
        grid=(N // _BM,),
        in_specs=[sp["p"], sp["h"], sp["d"], sp["b"]],
        out_specs=sp["h"],
        out_shape=jax.ShapeDtypeStruct((N, D), jnp.float32),
    )(part, h2p, degp, b2)


def kernel(x, edge_index, W1, b1, W2, b2):
    src_r = edge_index[0].astype(jnp.int32).reshape(NW, NCHUNK, CH)
    dst_r = edge_index[1].astype(jnp.int32).reshape(NW, NCHUNK, CH)
    ones128 = jnp.ones((CH, DEGW), jnp.float32)
    zer128 = jnp.zeros((CH, D), jnp.float32)

    degp = _deg(dst_r, ones128, zer128)
    h1p = _tc1(x, W1.T, degp)
    part1 = _scatter(h1p, src_r, dst_r, zer128)
    h2p = _tc_mid(part1, h1p, degp, b1.reshape(1, D), W2.T)
    part2 = _scatter(h2p, src_r, dst_r, zer128)
    return _tc_fin(part2, h2p, degp, b2.reshape(1, D))
